# Initial kernel scaffold; baseline (speedup 1.0000x reference)
import jax
import jax.numpy as jnp
from jax.experimental import pallas as pl


def kernel(p, x, q, W1, g1, b1, W2, g2, b2, Wt1, gt1, bt1, Wt2, gt2, bt2, Wt3, Wf, gf, bf):
    raise NotImplementedError("write your pallas kernel here")


# speedup vs baseline: 9.0177x; 9.0177x over previous
import jax
import jax.numpy as jnp
from jax.experimental import pallas as pl
from jax.experimental.pallas import tpu as pltpu
from jax.experimental.pallas import tpu_sc as plsc


_EPS = 1e-5
_SELU_ALPHA = 1.6732632423543772
_SELU_SCALE = 1.0507009873554805


def _selu(v):
    return _SELU_SCALE * jnp.where(v > 0, v, _SELU_ALPHA * (jnp.exp(v) - 1.0))


def _topk_kernel(p_ref, q_ref, idx_ref, ph_ref, *, n, k):
    b = pl.program_id(0)
    p = p_ref[0]
    q = q_ref[0]
    tm = q.shape[0]
    sp = jnp.sum(p * p, axis=0, keepdims=True)
    sq = jnp.sum(q * q, axis=1, keepdims=True)
    qb = q.astype(jnp.bfloat16).astype(jnp.float32)
    pb = p.astype(jnp.bfloat16).astype(jnp.float32)
    dot = (qb[:, 0:1] * pb[0:1, :]
           + qb[:, 1:2] * pb[1:2, :]
           + qb[:, 2:3] * pb[2:3, :])
    d = (sq + sp) - 2.0 * dot
    iota = jax.lax.broadcasted_iota(jnp.int32, d.shape, 1)
    iota_k = jax.lax.broadcasted_iota(jnp.int32, (tm, k), 1)
    base = b * n
    ph_ref[0] = jnp.zeros_like(ph_ref[0])
    zk_i = jnp.zeros((tm, k), jnp.int32)
    zk_f = jnp.zeros((tm, k), jnp.float32)

    def body(j, carry):
        dd, ia, cx, cy, cz = carry
        mv = jnp.min(dd, axis=1, keepdims=True)
        am = jnp.min(jnp.where(dd == mv, iota, jnp.int32(n)), axis=1,
                     keepdims=True)
        ohb = iota == am
        cs = [jnp.sum(jnp.where(ohb, p[c:c + 1, :], 0.0), axis=1,
                      keepdims=True) - q[:, c:c + 1] for c in range(3)]
        sel = iota_k == j
        ia = jnp.where(sel, am + base, ia)
        cx = jnp.where(sel, cs[0], cx)
        cy = jnp.where(sel, cs[1], cy)
        cz = jnp.where(sel, cs[2], cz)
        dd = jnp.where(ohb, jnp.float32(jnp.inf), dd)
        return dd, ia, cx, cy, cz

    _, ia, cx, cy, cz = jax.lax.fori_loop(
        0, k, body, (d, zk_i, zk_f, zk_f, zk_f))
    idx_ref[0] = ia
    ph_ref[0, :, :, 0:1] = cx[:, :, None]
    ph_ref[0, :, :, 1:2] = cy[:, :, None]
    ph_ref[0, :, :, 2:3] = cz[:, :, None]


def _run_topk(p, q, tm, k):
    bb, _, n = p.shape
    m = q.shape[1]
    return pl.pallas_call(
        lambda pr, qr, ir, phr: _topk_kernel(pr, qr, ir, phr, n=n, k=k),
        grid=(bb, m // tm),
        in_specs=[
            pl.BlockSpec((1, 3, n), lambda b, i: (b, 0, 0)),
            pl.BlockSpec((1, tm, 3), lambda b, i: (b, i, 0)),
        ],
        out_specs=[
            pl.BlockSpec((1, tm, k), lambda b, i: (b, i, 0)),
            pl.BlockSpec((1, tm, k, 16), lambda b, i: (b, i, 0, 0)),
        ],
        out_shape=[
            jax.ShapeDtypeStruct((bb, m, k), jnp.int32),
            jax.ShapeDtypeStruct((bb, m, k, 16), jnp.float32),
        ],
    )(p, q)


def _sc_gather(data, idx_flat, win):
    n_idx = idx_flat.shape[1]
    width = data.shape[1]
    mesh = plsc.VectorSubcoreMesh(core_axis_name="c", subcore_axis_name="s")

    @pl.kernel(out_type=jax.ShapeDtypeStruct((n_idx, width), data.dtype),
               mesh=mesh)
    def gk(x_hbm, i_hbm, o_hbm):
        def body(i_vmem, o_vmem):
            pltpu.sync_copy(x_hbm.at[i_vmem.at[0]], o_vmem)

        pltpu.emit_pipeline(
            body,
            grid=(n_idx // win,),
            in_specs=[pl.BlockSpec((1, win), index_map=lambda i: (0, i))],
            out_specs=[pl.BlockSpec((win, width), index_map=lambda i: (i, 0))],
            core_axis_name=("c", "s"),
            dimension_semantics=(pltpu.PARALLEL,),
        )(i_hbm, o_hbm)

    return gk(data, idx_flat)


def _moments_kernel(ps_ref, g_ref, s_ref, *, k):
    i = pl.program_id(0)
    cols = [ps_ref[:, j, :] for j in range(k)]
    cat = jnp.concatenate(cols, axis=1)
    g = jax.lax.dot_general(cat, cat, (((0,), (0,)), ((), ())),
                            preferred_element_type=jnp.float32)
    s = jnp.sum(cat, axis=0, keepdims=True)
    srow = jnp.concatenate(
        [s, jnp.zeros((7, s.shape[1]), jnp.float32)], axis=0)

    @pl.when(i == 0)
    def _():
        g_ref[...] = g
        s_ref[...] = srow

    @pl.when(i > 0)
    def _():
        g_ref[...] += g
        s_ref[...] += srow


def _run_moments(ph3, tmm, k):
    bm = ph3.shape[0]
    w = 16 * k
    return pl.pallas_call(
        lambda a, c, d: _moments_kernel(a, c, d, k=k),
        grid=(bm // tmm,),
        in_specs=[
            pl.BlockSpec((tmm, k, 16), lambda i: (i, 0, 0)),
        ],
        out_specs=[
            pl.BlockSpec((w, w), lambda i: (0, 0)),
            pl.BlockSpec((8, w), lambda i: (0, 0)),
        ],
        out_shape=[
            jax.ShapeDtypeStruct((w, w), jnp.float32),
            jax.ShapeDtypeStruct((8, w), jnp.float32),
        ],
    )(ph3)


def _mlp_kernel(ps_ref, w1q_ref, b1_ref, w2t_ref, wt1p_ref,
                bt1_ref, wt2t_ref, h2_ref, t2_ref, st_ref, *, k, mid):
    i = pl.program_id(0)
    tm2 = ps_ref.shape[0]

    phf = ps_ref[...].reshape(tm2 * k, 16)
    h1f = jax.lax.dot_general(phf, w1q_ref[...], (((1,), (0,)), ((), ())),
                              preferred_element_type=jnp.float32)
    h1f = _selu(h1f + b1_ref[...])
    h2f = jax.lax.dot_general(h1f, w2t_ref[...], (((1,), (0,)), ((), ())),
                              preferred_element_type=jnp.float32)
    h2_ref[...] = h2f.reshape(tm2, k, mid)

    t1 = bt1_ref[...]
    for j in range(k):
        t1 = t1 + jax.lax.dot_general(
            ps_ref[:, j, :], wt1p_ref[j], (((1,), (0,)), ((), ())),
            preferred_element_type=jnp.float32)
    t1 = _selu(t1)
    t2 = jax.lax.dot_general(t1, wt2t_ref[...], (((1,), (0,)), ((), ())),
                             preferred_element_type=jnp.float32)
    t2_ref[...] = t2

    kk = t2.shape[1]
    pad = jnp.zeros((1, kk), jnp.float32)
    row = jnp.concatenate([
        jnp.sum(h2f, axis=0, keepdims=True),
        jnp.sum(h2f * h2f, axis=0, keepdims=True),
        jnp.sum(t2, axis=0, keepdims=True),
        jnp.sum(t2 * t2, axis=0, keepdims=True),
        pad, pad, pad, pad], axis=0)

    @pl.when(i == 0)
    def _():
        st_ref[...] = row

    @pl.when(i > 0)
    def _():
        st_ref[...] += row


def _run_mlps(ph3, w1q, b1e, w2t, wt1p, bt1e, wt2t, tm2, k, mid):
    bm = ph3.shape[0]
    kk = wt2t.shape[1]
    return pl.pallas_call(
        lambda *a: _mlp_kernel(*a, k=k, mid=mid),
        grid=(bm // tm2,),
        in_specs=[
            pl.BlockSpec((tm2, k, 16), lambda i: (i, 0, 0)),
            pl.BlockSpec((16, mid), lambda i: (0, 0)),
            pl.BlockSpec((1, mid), lambda i: (0, 0)),
            pl.BlockSpec((mid, mid), lambda i: (0, 0)),
            pl.BlockSpec((k, 16, kk), lambda i: (0, 0, 0)),
            pl.BlockSpec((1, kk), lambda i: (0, 0)),
            pl.BlockSpec((kk, kk), lambda i: (0, 0)),
        ],
        out_specs=[
            pl.BlockSpec((tm2, k, mid), lambda i: (i, 0, 0)),
            pl.BlockSpec((tm2, kk), lambda i: (i, 0)),
            pl.BlockSpec((8, kk), lambda i: (0, 0)),
        ],
        out_shape=[
            jax.ShapeDtypeStruct((bm, k, mid), jnp.float32),
            jax.ShapeDtypeStruct((bm, kk), jnp.float32),
            jax.ShapeDtypeStruct((8, kk), jnp.float32),
        ],
    )(ph3, w1q, b1e, w2t, wt1p, bt1e, wt2t)


def _final_kernel(h2_ref, t2_ref, xs_ref, p2_ref, wt3t_ref, wfr_ref,
                  out_ref, st_ref, *, k, cout):
    b = pl.program_id(0)
    i = pl.program_id(1)
    a2 = p2_ref[0:1, :]
    c2 = p2_ref[1:2, :]
    at2 = p2_ref[2:3, :]
    ct2 = p2_ref[3:4, :]

    hh = _selu(h2_ref[...] * a2[None] + c2[None])
    tt = _selu(t2_ref[...] * at2 + ct2)
    t3 = jax.lax.dot_general(tt, wt3t_ref[...], (((1,), (0,)), ((), ())),
                             preferred_element_type=jnp.float32)
    xh = jnp.concatenate([hh, xs_ref[...]], axis=2)
    tm3 = xh.shape[0]

    acc = jnp.zeros((cout, tm3), jnp.float32)
    for kk_ in range(k):
        xm = t3[:, k * kk_:k * kk_ + 1] * xh[:, 0, :]
        for j in range(1, k):
            xm = xm + t3[:, k * kk_ + j:k * kk_ + j + 1] * xh[:, j, :]
        acc = acc + jax.lax.dot_general(
            wfr_ref[kk_], xm, (((0,), (1,)), ((), ())),
            preferred_element_type=jnp.float32)
    out_ref[0] = acc

    row = jnp.concatenate([
        jnp.sum(acc, axis=1, keepdims=True),
        jnp.sum(acc * acc, axis=1, keepdims=True),
        jnp.zeros((cout, 6), jnp.float32)], axis=1)

    first = jnp.logical_and(b == 0, i == 0)

    @pl.when(first)
    def _():
        st_ref[...] = row

    @pl.when(jnp.logical_not(first))
    def _():
        st_ref[...] += row


def _run_final(h2raw, t2raw, x_sel3, p2, wt3t, wfr, bb, m, tm3, k, cout):
    cin = x_sel3.shape[2]
    mid = h2raw.shape[2]
    kk = t2raw.shape[1]
    nt = m // tm3
    return pl.pallas_call(
        lambda *a: _final_kernel(*a, k=k, cout=cout),
        grid=(bb, nt),
        in_specs=[
            pl.BlockSpec((tm3, k, mid), lambda b, i: (b * nt + i, 0, 0)),
            pl.BlockSpec((tm3, kk), lambda b, i: (b * nt + i, 0)),
            pl.BlockSpec((tm3, k, cin), lambda b, i: (b * nt + i, 0, 0)),
            pl.BlockSpec((8, kk), lambda b, i: (0, 0)),
            pl.BlockSpec((kk, kk), lambda b, i: (0, 0)),
            pl.BlockSpec((k, mid + cin, cout), lambda b, i: (0, 0, 0)),
        ],
        out_specs=[
            pl.BlockSpec((1, cout, tm3), lambda b, i: (b, 0, i)),
            pl.BlockSpec((cout, 8), lambda b, i: (0, 0)),
        ],
        out_shape=[
            jax.ShapeDtypeStruct((bb, cout, m), jnp.float32),
            jax.ShapeDtypeStruct((cout, 8), jnp.float32),
        ],
    )(h2raw, t2raw, x_sel3, p2, wt3t, wfr)


def _bnout_kernel(o_ref, pf_ref, out_ref):
    af = pf_ref[:, 0:1]
    cf = pf_ref[:, 1:2]
    out_ref[0] = _selu(o_ref[0] * af + cf)


def _run_bnout(oraw, pf, tm4):
    bb, cout, m = oraw.shape
    return pl.pallas_call(
        _bnout_kernel,
        grid=(bb, m // tm4),
        in_specs=[
            pl.BlockSpec((1, cout, tm4), lambda b, i: (b, 0, i)),
            pl.BlockSpec((cout, 8), lambda b, i: (0, 0)),
        ],
        out_specs=pl.BlockSpec((1, cout, tm4), lambda b, i: (b, 0, i)),
        out_shape=jax.ShapeDtypeStruct((bb, cout, m), jnp.float32),
    )(oraw, pf)


def kernel(p, x, q, W1, g1, b1, W2, g2, b2, Wt1, gt1, bt1, Wt2, gt2, bt2,
           Wt3, Wf, gf, bf):
    bb, _, n = p.shape
    m = q.shape[1]
    cin = x.shape[1]
    cout, _, _, k = Wf.shape
    mid = W1.shape[0]
    kk = Wt1.shape[0]
    bm = bb * m
    bmk = bm * k

    tm = min(256, m)
    tmm = min(1024, bm)
    tm2 = min(512, bm)
    tm3 = min(512, m)
    tm4 = min(1024, m)

    idx, ph4 = _run_topk(p, q, tm, k)
    idx_flat = idx.reshape(1, bmk)
    ph3 = ph4.reshape(bm, k, 16)

    xt = jnp.transpose(x, (0, 2, 1)).reshape(bb * n, cin)
    x_sel = _sc_gather(xt, idx_flat, 128)
    x_sel3 = x_sel.reshape(bm, k, cin)

    g128, s128 = _run_moments(ph3, tmm, k)
    s128 = s128[0]
    g4 = g128.reshape(k, 16, k, 16)[:, 0:3, :, 0:3]
    s2d = s128.reshape(k, 16)[:, 0:3]

    w1m = W1.reshape(mid, 3)
    mu3 = jnp.sum(s2d, axis=0) / bmk
    s3 = jnp.einsum('iaib->ab', g4) / bmk
    mean1 = w1m @ mu3
    e2 = jnp.sum((w1m @ s3) * w1m, axis=1)
    var1 = jnp.maximum(e2 - mean1 * mean1, 0.0)
    a1 = g1 / jnp.sqrt(var1 + _EPS)
    c1 = b1 - a1 * mean1
    w1q = jnp.pad((w1m.T * a1[None, :]), ((0, 13), (0, 0)))
    b1e = c1.reshape(1, mid)

    wt1sq = Wt1[:, :, 0, :]
    wt1km = jnp.transpose(wt1sq, (0, 2, 1)).reshape(kk, 3 * k)
    mu24 = (s2d / bm).reshape(3 * k)
    m24 = jnp.transpose(g4, (0, 1, 2, 3)).reshape(k, 3, k, 3)
    m24 = jnp.reshape(m24, (3 * k, 3 * k)) / bm
    meant1 = wt1km @ mu24
    e2t = jnp.sum((wt1km @ m24) * wt1km, axis=1)
    vart1 = jnp.maximum(e2t - meant1 * meant1, 0.0)
    at1 = gt1 / jnp.sqrt(vart1 + _EPS)
    ct1 = bt1 - at1 * meant1
    wt1p = jnp.pad(jnp.transpose(wt1sq, (2, 1, 0)) * at1[None, None, :],
                   ((0, 0), (0, 13), (0, 0)))
    bt1e = ct1.reshape(1, kk)

    w2t = W2[:, :, 0, 0].T
    wt2t = Wt2[:, :, 0, 0].T
    h2raw, t2raw, st2 = _run_mlps(ph3, w1q, b1e, w2t, wt1p,
                                  bt1e, wt2t, tm2, k, mid)

    mean2 = st2[0] / bmk
    var2 = jnp.maximum(st2[1] / bmk - mean2 * mean2, 0.0)
    a2 = g2 / jnp.sqrt(var2 + _EPS)
    c2 = b2 - a2 * mean2
    meant2 = st2[2] / bm
    vart2 = jnp.maximum(st2[3] / bm - meant2 * meant2, 0.0)
    at2 = gt2 / jnp.sqrt(vart2 + _EPS)
    ct2 = bt2 - at2 * meant2
    p2 = jnp.stack([
        jnp.pad(a2, (0, kk - mid)), jnp.pad(c2, (0, kk - mid)),
        at2, ct2,
        jnp.zeros((kk,)), jnp.zeros((kk,)), jnp.zeros((kk,)),
        jnp.zeros((kk,))], axis=0)
    p2 = p2.astype(jnp.float32)

    wt3t = Wt3[:, :, 0, 0].T
    wfr = jnp.transpose(Wf[:, :, 0, :], (2, 1, 0))
    oraw, stf = _run_final(h2raw, t2raw, x_sel3, p2, wt3t, wfr,
                           bb, m, tm3, k, cout)

    meanf = stf[:, 0] / bm
    varf = jnp.maximum(stf[:, 1] / bm - meanf * meanf, 0.0)
    af = gf / jnp.sqrt(varf + _EPS)
    cf = bf - af * meanf
    pf = jnp.concatenate([af.reshape(cout, 1), cf.reshape(cout, 1),
                          jnp.zeros((cout, 6), jnp.float32)], axis=1)

    out = _run_bnout(oraw, pf, tm4)
    q_out = jnp.transpose(q, (0, 2, 1))
    return (q_out, out)


# speedup vs baseline: 12.9465x; 1.4357x over previous
import jax
import jax.numpy as jnp
from jax.experimental import pallas as pl
from jax.experimental.pallas import tpu as pltpu
from jax.experimental.pallas import tpu_sc as plsc


_EPS = 1e-5
_SELU_ALPHA = 1.6732632423543772
_SELU_SCALE = 1.0507009873554805


def _selu(v):
    return _SELU_SCALE * jnp.where(v > 0, v, _SELU_ALPHA * (jnp.exp(v) - 1.0))


def _topk_kernel(p_ref, q_ref, idx_ref, *, n, k):
    b = pl.program_id(0)
    p = p_ref[0]
    q = q_ref[0]
    tm = q.shape[0]
    sp = jnp.sum(p * p, axis=0, keepdims=True)
    sq = jnp.sum(q * q, axis=1, keepdims=True)
    qb = q.astype(jnp.bfloat16).astype(jnp.float32)
    pb = p.astype(jnp.bfloat16).astype(jnp.float32)
    dot = (qb[:, 0:1] * pb[0:1, :]
           + qb[:, 1:2] * pb[1:2, :]
           + qb[:, 2:3] * pb[2:3, :])
    d = (sq + sp) - 2.0 * dot
    iota = jax.lax.broadcasted_iota(jnp.int32, d.shape, 1)
    iota_k = jax.lax.broadcasted_iota(jnp.int32, (tm, k), 1)
    base = b * n
    zk_i = jnp.zeros((tm, k), jnp.int32)

    def body(j, carry):
        dd, ia = carry
        mv = jnp.min(dd, axis=1, keepdims=True)
        am = jnp.min(jnp.where(dd == mv, iota, jnp.int32(n)), axis=1,
                     keepdims=True)
        ia = jnp.where(iota_k == j, am + base, ia)
        dd = jnp.where(iota == am, jnp.float32(jnp.inf), dd)
        return dd, ia

    _, ia = jax.lax.fori_loop(0, k, body, (d, zk_i))
    idx_ref[0] = ia


def _run_topk(p, q, tm, k):
    bb, _, n = p.shape
    m = q.shape[1]
    return pl.pallas_call(
        lambda pr, qr, ir: _topk_kernel(pr, qr, ir, n=n, k=k),
        grid=(bb, m // tm),
        in_specs=[
            pl.BlockSpec((1, 3, n), lambda b, i: (b, 0, 0)),
            pl.BlockSpec((1, tm, 3), lambda b, i: (b, i, 0)),
        ],
        out_specs=pl.BlockSpec((1, tm, k), lambda b, i: (b, i, 0)),
        out_shape=jax.ShapeDtypeStruct((bb, m, k), jnp.int32),
    )(p, q)


def _sc_gather(data, idx_flat, win):
    n_idx = idx_flat.shape[1]
    width = data.shape[1]
    mesh = plsc.VectorSubcoreMesh(core_axis_name="c", subcore_axis_name="s")

    @pl.kernel(out_type=jax.ShapeDtypeStruct((n_idx, width), data.dtype),
               mesh=mesh)
    def gk(x_hbm, i_hbm, o_hbm):
        def body(i_vmem, o_vmem):
            pltpu.sync_copy(x_hbm.at[i_vmem.at[0]], o_vmem)

        pltpu.emit_pipeline(
            body,
            grid=(n_idx // win,),
            in_specs=[pl.BlockSpec((1, win), index_map=lambda i: (0, i))],
            out_specs=[pl.BlockSpec((win, width), index_map=lambda i: (i, 0))],
            core_axis_name=("c", "s"),
            dimension_semantics=(pltpu.PARALLEL,),
        )(i_hbm, o_hbm)

    return gk(data, idx_flat)


def _moments_kernel(ps_ref, q_ref, g_ref, s_ref, *, k):
    i = pl.program_id(0)
    q16 = q_ref[...]
    cols = [ps_ref[:, j, 0:16] - q16 for j in range(k)]
    cat = jnp.concatenate(cols, axis=1)
    g = jax.lax.dot_general(cat, cat, (((0,), (0,)), ((), ())),
                            preferred_element_type=jnp.float32)
    s = jnp.sum(cat, axis=0, keepdims=True)
    srow = jnp.concatenate(
        [s, jnp.zeros((7, s.shape[1]), jnp.float32)], axis=0)

    @pl.when(i == 0)
    def _():
        g_ref[...] = g
        s_ref[...] = srow

    @pl.when(i > 0)
    def _():
        g_ref[...] += g
        s_ref[...] += srow


def _run_moments(ps3, q16, tmm, k):
    bm = ps3.shape[0]
    w = 16 * k
    return pl.pallas_call(
        lambda a, b, c, d: _moments_kernel(a, b, c, d, k=k),
        grid=(bm // tmm,),
        in_specs=[
            pl.BlockSpec((tmm, k, 128), lambda i: (i, 0, 0)),
            pl.BlockSpec((tmm, 16), lambda i: (i, 0)),
        ],
        out_specs=[
            pl.BlockSpec((w, w), lambda i: (0, 0)),
            pl.BlockSpec((8, w), lambda i: (0, 0)),
        ],
        out_shape=[
            jax.ShapeDtypeStruct((w, w), jnp.float32),
            jax.ShapeDtypeStruct((8, w), jnp.float32),
        ],
    )(ps3, q16)


def _mlp_kernel(ps_ref, q_ref, w1q_ref, b1_ref, w2t_ref, wt1p_ref,
                bt1_ref, wt2t_ref, h2_ref, t2_ref, st_ref, *, k, mid):
    i = pl.program_id(0)
    tm2 = ps_ref.shape[0]
    ph3 = ps_ref[:, :, 0:16] - q_ref[...][:, None, :]

    phf = ph3.reshape(tm2 * k, 16)
    h1f = jax.lax.dot_general(phf, w1q_ref[...], (((1,), (0,)), ((), ())),
                              preferred_element_type=jnp.float32)
    h1f = _selu(h1f + b1_ref[...])
    h2f = jax.lax.dot_general(h1f, w2t_ref[...], (((1,), (0,)), ((), ())),
                              preferred_element_type=jnp.float32)
    h2_ref[...] = h2f.reshape(tm2, k, mid)

    t1 = bt1_ref[...]
    for j in range(k):
        t1 = t1 + jax.lax.dot_general(
            ph3[:, j, :], wt1p_ref[j], (((1,), (0,)), ((), ())),
            preferred_element_type=jnp.float32)
    t1 = _selu(t1)
    t2 = jax.lax.dot_general(t1, wt2t_ref[...], (((1,), (0,)), ((), ())),
                             preferred_element_type=jnp.float32)
    t2_ref[...] = t2

    kk = t2.shape[1]
    pad = jnp.zeros((1, kk), jnp.float32)
    row = jnp.concatenate([
        jnp.sum(h2f, axis=0, keepdims=True),
        jnp.sum(h2f * h2f, axis=0, keepdims=True),
        jnp.sum(t2, axis=0, keepdims=True),
        jnp.sum(t2 * t2, axis=0, keepdims=True),
        pad, pad, pad, pad], axis=0)

    @pl.when(i == 0)
    def _():
        st_ref[...] = row

    @pl.when(i > 0)
    def _():
        st_ref[...] += row


def _run_mlps(ps3, q16, w1q, b1e, w2t, wt1p, bt1e, wt2t, tm2, k, mid):
    bm = ps3.shape[0]
    kk = wt2t.shape[1]
    return pl.pallas_call(
        lambda *a: _mlp_kernel(*a, k=k, mid=mid),
        grid=(bm // tm2,),
        in_specs=[
            pl.BlockSpec((tm2, k, 128), lambda i: (i, 0, 0)),
            pl.BlockSpec((tm2, 16), lambda i: (i, 0)),
            pl.BlockSpec((16, mid), lambda i: (0, 0)),
            pl.BlockSpec((1, mid), lambda i: (0, 0)),
            pl.BlockSpec((mid, mid), lambda i: (0, 0)),
            pl.BlockSpec((k, 16, kk), lambda i: (0, 0, 0)),
            pl.BlockSpec((1, kk), lambda i: (0, 0)),
            pl.BlockSpec((kk, kk), lambda i: (0, 0)),
        ],
        out_specs=[
            pl.BlockSpec((tm2, k, mid), lambda i: (i, 0, 0)),
            pl.BlockSpec((tm2, kk), lambda i: (i, 0)),
            pl.BlockSpec((8, kk), lambda i: (0, 0)),
        ],
        out_shape=[
            jax.ShapeDtypeStruct((bm, k, mid), jnp.float32),
            jax.ShapeDtypeStruct((bm, kk), jnp.float32),
            jax.ShapeDtypeStruct((8, kk), jnp.float32),
        ],
    )(ps3, q16, w1q, b1e, w2t, wt1p, bt1e, wt2t)


def _final_kernel(h2_ref, t2_ref, xs_ref, p2_ref, wt3t_ref, wfr_ref,
                  out_ref, st_ref, *, k, cout):
    b = pl.program_id(0)
    i = pl.program_id(1)
    a2 = p2_ref[0:1, :]
    c2 = p2_ref[1:2, :]
    at2 = p2_ref[2:3, :]
    ct2 = p2_ref[3:4, :]

    hh = _selu(h2_ref[...] * a2[None] + c2[None])
    tt = _selu(t2_ref[...] * at2 + ct2)
    t3 = jax.lax.dot_general(tt, wt3t_ref[...], (((1,), (0,)), ((), ())),
                             preferred_element_type=jnp.float32)
    xh = jnp.concatenate([hh, xs_ref[...]], axis=2)
    tm3 = xh.shape[0]

    acc = jnp.zeros((cout, tm3), jnp.float32)
    for kk_ in range(k):
        xm = t3[:, k * kk_:k * kk_ + 1] * xh[:, 0, :]
        for j in range(1, k):
            xm = xm + t3[:, k * kk_ + j:k * kk_ + j + 1] * xh[:, j, :]
        acc = acc + jax.lax.dot_general(
            wfr_ref[kk_], xm, (((0,), (1,)), ((), ())),
            preferred_element_type=jnp.float32)
    out_ref[0] = acc

    row = jnp.concatenate([
        jnp.sum(acc, axis=1, keepdims=True),
        jnp.sum(acc * acc, axis=1, keepdims=True),
        jnp.zeros((cout, 6), jnp.float32)], axis=1)

    first = jnp.logical_and(b == 0, i == 0)

    @pl.when(first)
    def _():
        st_ref[...] = row

    @pl.when(jnp.logical_not(first))
    def _():
        st_ref[...] += row


def _run_final(h2raw, t2raw, x_sel3, p2, wt3t, wfr, bb, m, tm3, k, cout):
    cin = x_sel3.shape[2]
    mid = h2raw.shape[2]
    kk = t2raw.shape[1]
    nt = m // tm3
    return pl.pallas_call(
        lambda *a: _final_kernel(*a, k=k, cout=cout),
        grid=(bb, nt),
        in_specs=[
            pl.BlockSpec((tm3, k, mid), lambda b, i: (b * nt + i, 0, 0)),
            pl.BlockSpec((tm3, kk), lambda b, i: (b * nt + i, 0)),
            pl.BlockSpec((tm3, k, cin), lambda b, i: (b * nt + i, 0, 0)),
            pl.BlockSpec((8, kk), lambda b, i: (0, 0)),
            pl.BlockSpec((kk, kk), lambda b, i: (0, 0)),
            pl.BlockSpec((k, mid + cin, cout), lambda b, i: (0, 0, 0)),
        ],
        out_specs=[
            pl.BlockSpec((1, cout, tm3), lambda b, i: (b, 0, i)),
            pl.BlockSpec((cout, 8), lambda b, i: (0, 0)),
        ],
        out_shape=[
            jax.ShapeDtypeStruct((bb, cout, m), jnp.float32),
            jax.ShapeDtypeStruct((cout, 8), jnp.float32),
        ],
    )(h2raw, t2raw, x_sel3, p2, wt3t, wfr)


def _bnout_kernel(o_ref, pf_ref, out_ref):
    af = pf_ref[:, 0:1]
    cf = pf_ref[:, 1:2]
    out_ref[0] = _selu(o_ref[0] * af + cf)


def _run_bnout(oraw, pf, tm4):
    bb, cout, m = oraw.shape
    return pl.pallas_call(
        _bnout_kernel,
        grid=(bb, m // tm4),
        in_specs=[
            pl.BlockSpec((1, cout, tm4), lambda b, i: (b, 0, i)),
            pl.BlockSpec((cout, 8), lambda b, i: (0, 0)),
        ],
        out_specs=pl.BlockSpec((1, cout, tm4), lambda b, i: (b, 0, i)),
        out_shape=jax.ShapeDtypeStruct((bb, cout, m), jnp.float32),
    )(oraw, pf)


def kernel(p, x, q, W1, g1, b1, W2, g2, b2, Wt1, gt1, bt1, Wt2, gt2, bt2,
           Wt3, Wf, gf, bf):
    bb, _, n = p.shape
    m = q.shape[1]
    cin = x.shape[1]
    cout, _, _, k = Wf.shape
    mid = W1.shape[0]
    kk = Wt1.shape[0]
    bm = bb * m
    bmk = bm * k

    tm = min(256, m)
    tmm = min(1024, bm)
    tm2 = min(512, bm)
    tm3 = min(512, m)
    tm4 = min(1024, m)

    idx = _run_topk(p, q, tm, k)
    idx_flat = idx.reshape(1, bmk)

    ptp = jnp.pad(jnp.transpose(p, (0, 2, 1)),
                  ((0, 0), (0, 0), (0, 125))).reshape(bb * n, 128)
    xt = jnp.transpose(x, (0, 2, 1)).reshape(bb * n, cin)
    p_sel = _sc_gather(ptp, idx_flat, 128)
    x_sel = _sc_gather(xt, idx_flat, 128)
    ps3 = p_sel.reshape(bm, k, 128)
    x_sel3 = x_sel.reshape(bm, k, cin)
    q16 = jnp.pad(q.reshape(bm, 3), ((0, 0), (0, 13)))

    g128, s128 = _run_moments(ps3, q16, tmm, k)
    s128 = s128[0]
    g4 = g128.reshape(k, 16, k, 16)[:, 0:3, :, 0:3]
    s2d = s128.reshape(k, 16)[:, 0:3]

    w1m = W1.reshape(mid, 3)
    mu3 = jnp.sum(s2d, axis=0) / bmk
    s3 = jnp.einsum('iaib->ab', g4) / bmk
    mean1 = w1m @ mu3
    e2 = jnp.sum((w1m @ s3) * w1m, axis=1)
    var1 = jnp.maximum(e2 - mean1 * mean1, 0.0)
    a1 = g1 / jnp.sqrt(var1 + _EPS)
    c1 = b1 - a1 * mean1
    w1q = jnp.pad((w1m.T * a1[None, :]), ((0, 13), (0, 0)))
    b1e = c1.reshape(1, mid)

    wt1sq = Wt1[:, :, 0, :]
    wt1km = jnp.transpose(wt1sq, (0, 2, 1)).reshape(kk, 3 * k)
    mu24 = (s2d / bm).reshape(3 * k)
    m24 = jnp.transpose(g4, (0, 1, 2, 3)).reshape(k, 3, k, 3)
    m24 = jnp.reshape(m24, (3 * k, 3 * k)) / bm
    meant1 = wt1km @ mu24
    e2t = jnp.sum((wt1km @ m24) * wt1km, axis=1)
    vart1 = jnp.maximum(e2t - meant1 * meant1, 0.0)
    at1 = gt1 / jnp.sqrt(vart1 + _EPS)
    ct1 = bt1 - at1 * meant1
    wt1p = jnp.pad(jnp.transpose(wt1sq, (2, 1, 0)) * at1[None, None, :],
                   ((0, 0), (0, 13), (0, 0)))
    bt1e = ct1.reshape(1, kk)

    w2t = W2[:, :, 0, 0].T
    wt2t = Wt2[:, :, 0, 0].T
    h2raw, t2raw, st2 = _run_mlps(ps3, q16, w1q, b1e, w2t, wt1p,
                                  bt1e, wt2t, tm2, k, mid)

    mean2 = st2[0] / bmk
    var2 = jnp.maximum(st2[1] / bmk - mean2 * mean2, 0.0)
    a2 = g2 / jnp.sqrt(var2 + _EPS)
    c2 = b2 - a2 * mean2
    meant2 = st2[2] / bm
    vart2 = jnp.maximum(st2[3] / bm - meant2 * meant2, 0.0)
    at2 = gt2 / jnp.sqrt(vart2 + _EPS)
    ct2 = bt2 - at2 * meant2
    p2 = jnp.stack([
        jnp.pad(a2, (0, kk - mid)), jnp.pad(c2, (0, kk - mid)),
        at2, ct2,
        jnp.zeros((kk,)), jnp.zeros((kk,)), jnp.zeros((kk,)),
        jnp.zeros((kk,))], axis=0)
    p2 = p2.astype(jnp.float32)

    wt3t = Wt3[:, :, 0, 0].T
    wfr = jnp.transpose(Wf[:, :, 0, :], (2, 1, 0))
    oraw, stf = _run_final(h2raw, t2raw, x_sel3, p2, wt3t, wfr,
                           bb, m, tm3, k, cout)

    meanf = stf[:, 0] / bm
    varf = jnp.maximum(stf[:, 1] / bm - meanf * meanf, 0.0)
    af = gf / jnp.sqrt(varf + _EPS)
    cf = bf - af * meanf
    pf = jnp.concatenate([af.reshape(cout, 1), cf.reshape(cout, 1),
                          jnp.zeros((cout, 6), jnp.float32)], axis=1)

    out = _run_bnout(oraw, pf, tm4)
    q_out = jnp.transpose(q, (0, 2, 1))
    return (q_out, out)


# speedup vs baseline: 13.7707x; 1.0637x over previous
import jax
import jax.numpy as jnp
from jax.experimental import pallas as pl
from jax.experimental.pallas import tpu as pltpu
from jax.experimental.pallas import tpu_sc as plsc


_EPS = 1e-5
_SELU_ALPHA = 1.6732632423543772
_SELU_SCALE = 1.0507009873554805


def _selu(v):
    return _SELU_SCALE * jnp.where(v > 0, v, _SELU_ALPHA * (jnp.exp(v) - 1.0))


def _topk_kernel(p_ref, q_ref, idx_ref, *, n, k):
    b = pl.program_id(0)
    p = p_ref[0]
    q = q_ref[0]
    tm = q.shape[0]
    sp = jnp.sum(p * p, axis=0, keepdims=True)
    sq = jnp.sum(q * q, axis=1, keepdims=True)
    qb = q.astype(jnp.bfloat16)
    pb = p.astype(jnp.bfloat16)
    dot = jax.lax.dot_general(qb, pb, (((1,), (0,)), ((), ())),
                              preferred_element_type=jnp.float32)
    d = (sq + sp) - 2.0 * dot
    iota = jax.lax.broadcasted_iota(jnp.int32, d.shape, 1)
    iota_k = jax.lax.broadcasted_iota(jnp.int32, (tm, k), 1)
    base = b * n
    zk_i = jnp.zeros((tm, k), jnp.int32)

    def body(j, carry):
        dd, ia = carry
        am = jnp.argmin(dd, axis=1, keepdims=True).astype(jnp.int32)
        ia = jnp.where(iota_k == j, am + base, ia)
        dd = jnp.where(iota == am, jnp.float32(jnp.inf), dd)
        return dd, ia

    _, ia = jax.lax.fori_loop(0, k, body, (d, zk_i))
    idx_ref[0] = ia


def _run_topk(p, q, tm, k):
    bb, _, n = p.shape
    m = q.shape[1]
    return pl.pallas_call(
        lambda pr, qr, ir: _topk_kernel(pr, qr, ir, n=n, k=k),
        grid=(bb, m // tm),
        in_specs=[
            pl.BlockSpec((1, 3, n), lambda b, i: (b, 0, 0)),
            pl.BlockSpec((1, tm, 3), lambda b, i: (b, i, 0)),
        ],
        out_specs=pl.BlockSpec((1, tm, k), lambda b, i: (b, i, 0)),
        out_shape=jax.ShapeDtypeStruct((bb, m, k), jnp.int32),
    )(p, q)


def _sc_gather(data, idx_flat, win):
    n_idx = idx_flat.shape[1]
    width = data.shape[1]
    mesh = plsc.VectorSubcoreMesh(core_axis_name="c", subcore_axis_name="s")

    @pl.kernel(out_type=jax.ShapeDtypeStruct((n_idx, width), data.dtype),
               mesh=mesh)
    def gk(x_hbm, i_hbm, o_hbm):
        def body(i_vmem, o_vmem):
            pltpu.sync_copy(x_hbm.at[i_vmem.at[0]], o_vmem)

        pltpu.emit_pipeline(
            body,
            grid=(n_idx // win,),
            in_specs=[pl.BlockSpec((1, win), index_map=lambda i: (0, i))],
            out_specs=[pl.BlockSpec((win, width), index_map=lambda i: (i, 0))],
            core_axis_name=("c", "s"),
            dimension_semantics=(pltpu.PARALLEL,),
        )(i_hbm, o_hbm)

    return gk(data, idx_flat)


def _moments_kernel(ps_ref, q_ref, g_ref, s_ref, *, k):
    i = pl.program_id(0)
    q16 = q_ref[...]
    cols = [ps_ref[:, j, 0:16] - q16 for j in range(k)]
    cat = jnp.concatenate(cols, axis=1)
    g = jax.lax.dot_general(cat, cat, (((0,), (0,)), ((), ())),
                            preferred_element_type=jnp.float32)
    s = jnp.sum(cat, axis=0, keepdims=True)
    srow = jnp.concatenate(
        [s, jnp.zeros((7, s.shape[1]), jnp.float32)], axis=0)

    @pl.when(i == 0)
    def _():
        g_ref[...] = g
        s_ref[...] = srow

    @pl.when(i > 0)
    def _():
        g_ref[...] += g
        s_ref[...] += srow


def _run_moments(ps3, q16, tmm, k):
    bm = ps3.shape[0]
    w = 16 * k
    return pl.pallas_call(
        lambda a, b, c, d: _moments_kernel(a, b, c, d, k=k),
        grid=(bm // tmm,),
        in_specs=[
            pl.BlockSpec((tmm, k, 128), lambda i: (i, 0, 0)),
            pl.BlockSpec((tmm, 16), lambda i: (i, 0)),
        ],
        out_specs=[
            pl.BlockSpec((w, w), lambda i: (0, 0)),
            pl.BlockSpec((8, w), lambda i: (0, 0)),
        ],
        out_shape=[
            jax.ShapeDtypeStruct((w, w), jnp.float32),
            jax.ShapeDtypeStruct((8, w), jnp.float32),
        ],
    )(ps3, q16)


def _mlp_kernel(ps_ref, q_ref, w1q_ref, b1_ref, w2t_ref, wt1p_ref,
                bt1_ref, wt2t_ref, h2_ref, t2_ref, st_ref, *, k, mid):
    i = pl.program_id(0)
    tm2 = ps_ref.shape[0]
    ph3 = ps_ref[:, :, 0:16] - q_ref[...][:, None, :]

    phf = ph3.reshape(tm2 * k, 16)
    h1f = jax.lax.dot_general(phf, w1q_ref[...], (((1,), (0,)), ((), ())),
                              preferred_element_type=jnp.float32)
    h1f = _selu(h1f + b1_ref[...])
    h2f = jax.lax.dot_general(h1f, w2t_ref[...], (((1,), (0,)), ((), ())),
                              preferred_element_type=jnp.float32)
    h2_ref[...] = h2f.reshape(tm2, k, mid)

    t1 = bt1_ref[...]
    for j in range(k):
        t1 = t1 + jax.lax.dot_general(
            ph3[:, j, :], wt1p_ref[j], (((1,), (0,)), ((), ())),
            preferred_element_type=jnp.float32)
    t1 = _selu(t1)
    t2 = jax.lax.dot_general(t1, wt2t_ref[...], (((1,), (0,)), ((), ())),
                             preferred_element_type=jnp.float32)
    t2_ref[...] = t2

    kk = t2.shape[1]
    pad = jnp.zeros((1, kk), jnp.float32)
    row = jnp.concatenate([
        jnp.sum(h2f, axis=0, keepdims=True),
        jnp.sum(h2f * h2f, axis=0, keepdims=True),
        jnp.sum(t2, axis=0, keepdims=True),
        jnp.sum(t2 * t2, axis=0, keepdims=True),
        pad, pad, pad, pad], axis=0)

    @pl.when(i == 0)
    def _():
        st_ref[...] = row

    @pl.when(i > 0)
    def _():
        st_ref[...] += row


def _run_mlps(ps3, q16, w1q, b1e, w2t, wt1p, bt1e, wt2t, tm2, k, mid):
    bm = ps3.shape[0]
    kk = wt2t.shape[1]
    return pl.pallas_call(
        lambda *a: _mlp_kernel(*a, k=k, mid=mid),
        grid=(bm // tm2,),
        in_specs=[
            pl.BlockSpec((tm2, k, 128), lambda i: (i, 0, 0)),
            pl.BlockSpec((tm2, 16), lambda i: (i, 0)),
            pl.BlockSpec((16, mid), lambda i: (0, 0)),
            pl.BlockSpec((1, mid), lambda i: (0, 0)),
            pl.BlockSpec((mid, mid), lambda i: (0, 0)),
            pl.BlockSpec((k, 16, kk), lambda i: (0, 0, 0)),
            pl.BlockSpec((1, kk), lambda i: (0, 0)),
            pl.BlockSpec((kk, kk), lambda i: (0, 0)),
        ],
        out_specs=[
            pl.BlockSpec((tm2, k, mid), lambda i: (i, 0, 0)),
            pl.BlockSpec((tm2, kk), lambda i: (i, 0)),
            pl.BlockSpec((8, kk), lambda i: (0, 0)),
        ],
        out_shape=[
            jax.ShapeDtypeStruct((bm, k, mid), jnp.float32),
            jax.ShapeDtypeStruct((bm, kk), jnp.float32),
            jax.ShapeDtypeStruct((8, kk), jnp.float32),
        ],
    )(ps3, q16, w1q, b1e, w2t, wt1p, bt1e, wt2t)


def _final_kernel(h2_ref, t2_ref, xs_ref, p2_ref, wt3t_ref, wfr_ref,
                  out_ref, st_ref, *, k, cout):
    b = pl.program_id(0)
    i = pl.program_id(1)
    a2 = p2_ref[0:1, :]
    c2 = p2_ref[1:2, :]
    at2 = p2_ref[2:3, :]
    ct2 = p2_ref[3:4, :]

    hh = _selu(h2_ref[...] * a2[None] + c2[None])
    tt = _selu(t2_ref[...] * at2 + ct2)
    t3 = jax.lax.dot_general(tt, wt3t_ref[...], (((1,), (0,)), ((), ())),
                             preferred_element_type=jnp.float32)
    xh = jnp.concatenate([hh, xs_ref[...]], axis=2)
    tm3 = xh.shape[0]

    acc = jnp.zeros((cout, tm3), jnp.float32)
    for kk_ in range(k):
        xm = t3[:, k * kk_:k * kk_ + 1] * xh[:, 0, :]
        for j in range(1, k):
            xm = xm + t3[:, k * kk_ + j:k * kk_ + j + 1] * xh[:, j, :]
        acc = acc + jax.lax.dot_general(
            wfr_ref[kk_], xm, (((0,), (1,)), ((), ())),
            preferred_element_type=jnp.float32)
    out_ref[0] = acc

    row = jnp.concatenate([
        jnp.sum(acc, axis=1, keepdims=True),
        jnp.sum(acc * acc, axis=1, keepdims=True),
        jnp.zeros((cout, 6), jnp.float32)], axis=1)

    first = jnp.logical_and(b == 0, i == 0)

    @pl.when(first)
    def _():
        st_ref[...] = row

    @pl.when(jnp.logical_not(first))
    def _():
        st_ref[...] += row


def _run_final(h2raw, t2raw, x_sel3, p2, wt3t, wfr, bb, m, tm3, k, cout):
    cin = x_sel3.shape[2]
    mid = h2raw.shape[2]
    kk = t2raw.shape[1]
    nt = m // tm3
    return pl.pallas_call(
        lambda *a: _final_kernel(*a, k=k, cout=cout),
        grid=(bb, nt),
        in_specs=[
            pl.BlockSpec((tm3, k, mid), lambda b, i: (b * nt + i, 0, 0)),
            pl.BlockSpec((tm3, kk), lambda b, i: (b * nt + i, 0)),
            pl.BlockSpec((tm3, k, cin), lambda b, i: (b * nt + i, 0, 0)),
            pl.BlockSpec((8, kk), lambda b, i: (0, 0)),
            pl.BlockSpec((kk, kk), lambda b, i: (0, 0)),
            pl.BlockSpec((k, mid + cin, cout), lambda b, i: (0, 0, 0)),
        ],
        out_specs=[
            pl.BlockSpec((1, cout, tm3), lambda b, i: (b, 0, i)),
            pl.BlockSpec((cout, 8), lambda b, i: (0, 0)),
        ],
        out_shape=[
            jax.ShapeDtypeStruct((bb, cout, m), jnp.float32),
            jax.ShapeDtypeStruct((cout, 8), jnp.float32),
        ],
    )(h2raw, t2raw, x_sel3, p2, wt3t, wfr)


def _bnout_kernel(o_ref, pf_ref, out_ref):
    af = pf_ref[:, 0:1]
    cf = pf_ref[:, 1:2]
    out_ref[0] = _selu(o_ref[0] * af + cf)


def _run_bnout(oraw, pf, tm4):
    bb, cout, m = oraw.shape
    return pl.pallas_call(
        _bnout_kernel,
        grid=(bb, m // tm4),
        in_specs=[
            pl.BlockSpec((1, cout, tm4), lambda b, i: (b, 0, i)),
            pl.BlockSpec((cout, 8), lambda b, i: (0, 0)),
        ],
        out_specs=pl.BlockSpec((1, cout, tm4), lambda b, i: (b, 0, i)),
        out_shape=jax.ShapeDtypeStruct((bb, cout, m), jnp.float32),
    )(oraw, pf)


def kernel(p, x, q, W1, g1, b1, W2, g2, b2, Wt1, gt1, bt1, Wt2, gt2, bt2,
           Wt3, Wf, gf, bf):
    bb, _, n = p.shape
    m = q.shape[1]
    cin = x.shape[1]
    cout, _, _, k = Wf.shape
    mid = W1.shape[0]
    kk = Wt1.shape[0]
    bm = bb * m
    bmk = bm * k

    tm = min(256, m)
    tmm = min(1024, bm)
    tm2 = min(512, bm)
    tm3 = min(512, m)
    tm4 = min(1024, m)

    idx = _run_topk(p, q, tm, k)
    idx_flat = idx.reshape(1, bmk)

    ptp = jnp.pad(jnp.transpose(p, (0, 2, 1)),
                  ((0, 0), (0, 0), (0, 125))).reshape(bb * n, 128)
    xt = jnp.transpose(x, (0, 2, 1)).reshape(bb * n, cin)
    p_sel = _sc_gather(ptp, idx_flat, 128)
    x_sel = _sc_gather(xt, idx_flat, 128)
    ps3 = p_sel.reshape(bm, k, 128)
    x_sel3 = x_sel.reshape(bm, k, cin)
    q16 = jnp.pad(q.reshape(bm, 3), ((0, 0), (0, 13)))

    g128, s128 = _run_moments(ps3, q16, tmm, k)
    s128 = s128[0]
    g4 = g128.reshape(k, 16, k, 16)[:, 0:3, :, 0:3]
    s2d = s128.reshape(k, 16)[:, 0:3]

    w1m = W1.reshape(mid, 3)
    mu3 = jnp.sum(s2d, axis=0) / bmk
    s3 = jnp.einsum('iaib->ab', g4) / bmk
    mean1 = w1m @ mu3
    e2 = jnp.sum((w1m @ s3) * w1m, axis=1)
    var1 = jnp.maximum(e2 - mean1 * mean1, 0.0)
    a1 = g1 / jnp.sqrt(var1 + _EPS)
    c1 = b1 - a1 * mean1
    w1q = jnp.pad((w1m.T * a1[None, :]), ((0, 13), (0, 0)))
    b1e = c1.reshape(1, mid)

    wt1sq = Wt1[:, :, 0, :]
    wt1km = jnp.transpose(wt1sq, (0, 2, 1)).reshape(kk, 3 * k)
    mu24 = (s2d / bm).reshape(3 * k)
    m24 = jnp.transpose(g4, (0, 1, 2, 3)).reshape(k, 3, k, 3)
    m24 = jnp.reshape(m24, (3 * k, 3 * k)) / bm
    meant1 = wt1km @ mu24
    e2t = jnp.sum((wt1km @ m24) * wt1km, axis=1)
    vart1 = jnp.maximum(e2t - meant1 * meant1, 0.0)
    at1 = gt1 / jnp.sqrt(vart1 + _EPS)
    ct1 = bt1 - at1 * meant1
    wt1p = jnp.pad(jnp.transpose(wt1sq, (2, 1, 0)) * at1[None, None, :],
                   ((0, 0), (0, 13), (0, 0)))
    bt1e = ct1.reshape(1, kk)

    w2t = W2[:, :, 0, 0].T
    wt2t = Wt2[:, :, 0, 0].T
    h2raw, t2raw, st2 = _run_mlps(ps3, q16, w1q, b1e, w2t, wt1p,
                                  bt1e, wt2t, tm2, k, mid)

    mean2 = st2[0] / bmk
    var2 = jnp.maximum(st2[1] / bmk - mean2 * mean2, 0.0)
    a2 = g2 / jnp.sqrt(var2 + _EPS)
    c2 = b2 - a2 * mean2
    meant2 = st2[2] / bm
    vart2 = jnp.maximum(st2[3] / bm - meant2 * meant2, 0.0)
    at2 = gt2 / jnp.sqrt(vart2 + _EPS)
    ct2 = bt2 - at2 * meant2
    p2 = jnp.stack([
        jnp.pad(a2, (0, kk - mid)), jnp.pad(c2, (0, kk - mid)),
        at2, ct2,
        jnp.zeros((kk,)), jnp.zeros((kk,)), jnp.zeros((kk,)),
        jnp.zeros((kk,))], axis=0)
    p2 = p2.astype(jnp.float32)

    wt3t = Wt3[:, :, 0, 0].T
    wfr = jnp.transpose(Wf[:, :, 0, :], (2, 1, 0))
    oraw, stf = _run_final(h2raw, t2raw, x_sel3, p2, wt3t, wfr,
                           bb, m, tm3, k, cout)

    meanf = stf[:, 0] / bm
    varf = jnp.maximum(stf[:, 1] / bm - meanf * meanf, 0.0)
    af = gf / jnp.sqrt(varf + _EPS)
    cf = bf - af * meanf
    pf = jnp.concatenate([af.reshape(cout, 1), cf.reshape(cout, 1),
                          jnp.zeros((cout, 6), jnp.float32)], axis=1)

    out = _run_bnout(oraw, pf, tm4)
    q_out = jnp.transpose(q, (0, 2, 1))
    return (q_out, out)


# speedup vs baseline: 14.0436x; 1.0198x over previous
import jax
import jax.numpy as jnp
from jax.experimental import pallas as pl
from jax.experimental.pallas import tpu as pltpu
from jax.experimental.pallas import tpu_sc as plsc


_EPS = 1e-5
_SELU_ALPHA = 1.6732632423543772
_SELU_SCALE = 1.0507009873554805


def _selu(v):
    return _SELU_SCALE * jnp.where(v > 0, v, _SELU_ALPHA * (jnp.exp(v) - 1.0))


def _topk_kernel(p_ref, q_ref, idx_ref, *, n, k):
    b = pl.program_id(0)
    p = p_ref[0]
    q = q_ref[0]
    tm = q.shape[0]
    sp = jnp.sum(p * p, axis=0, keepdims=True)
    sq = jnp.sum(q * q, axis=1, keepdims=True)
    qb = q.astype(jnp.bfloat16)
    pb = p.astype(jnp.bfloat16)
    dot = jax.lax.dot_general(qb, pb, (((1,), (0,)), ((), ())),
                              preferred_element_type=jnp.float32)
    d = (sq + sp) - 2.0 * dot
    iota = jax.lax.broadcasted_iota(jnp.int32, d.shape, 1)
    iota_k = jax.lax.broadcasted_iota(jnp.int32, (tm, k), 1)
    base = b * n
    zk_i = jnp.zeros((tm, k), jnp.int32)

    def body(j, carry):
        dd, ia, am_prev = carry
        dd = jnp.where(iota == am_prev, jnp.float32(jnp.inf), dd)
        am = jnp.argmin(dd, axis=1, keepdims=True).astype(jnp.int32)
        ia = jnp.where(iota_k == j, am + base, ia)
        return dd, ia, am

    _, ia, _ = jax.lax.fori_loop(
        0, k, body, (d, zk_i, jnp.full((tm, 1), -1, jnp.int32)))
    idx_ref[0] = ia


def _run_topk(p, q, tm, k):
    bb, _, n = p.shape
    m = q.shape[1]
    return pl.pallas_call(
        lambda pr, qr, ir: _topk_kernel(pr, qr, ir, n=n, k=k),
        grid=(bb, m // tm),
        in_specs=[
            pl.BlockSpec((1, 3, n), lambda b, i: (b, 0, 0)),
            pl.BlockSpec((1, tm, 3), lambda b, i: (b, i, 0)),
        ],
        out_specs=pl.BlockSpec((1, tm, k), lambda b, i: (b, i, 0)),
        out_shape=jax.ShapeDtypeStruct((bb, m, k), jnp.int32),
    )(p, q)


def _sc_gather(data, idx_flat, win):
    n_idx = idx_flat.shape[1]
    width = data.shape[1]
    mesh = plsc.VectorSubcoreMesh(core_axis_name="c", subcore_axis_name="s")

    @pl.kernel(out_type=jax.ShapeDtypeStruct((n_idx, width), data.dtype),
               mesh=mesh)
    def gk(x_hbm, i_hbm, o_hbm):
        def body(i_vmem, o_vmem):
            pltpu.sync_copy(x_hbm.at[i_vmem.at[0]], o_vmem)

        pltpu.emit_pipeline(
            body,
            grid=(n_idx // win,),
            in_specs=[pl.BlockSpec((1, win), index_map=lambda i: (0, i))],
            out_specs=[pl.BlockSpec((win, width), index_map=lambda i: (i, 0))],
            core_axis_name=("c", "s"),
            dimension_semantics=(pltpu.PARALLEL,),
        )(i_hbm, o_hbm)

    return gk(data, idx_flat)


def _moments_kernel(ps_ref, q_ref, g_ref, s_ref, *, k):
    i = pl.program_id(0)
    q16 = q_ref[...]
    cols = [ps_ref[:, j, 0:16] - q16 for j in range(k)]
    cat = jnp.concatenate(cols, axis=1)
    g = jax.lax.dot_general(cat, cat, (((0,), (0,)), ((), ())),
                            preferred_element_type=jnp.float32)
    s = jnp.sum(cat, axis=0, keepdims=True)
    srow = jnp.concatenate(
        [s, jnp.zeros((7, s.shape[1]), jnp.float32)], axis=0)

    @pl.when(i == 0)
    def _():
        g_ref[...] = g
        s_ref[...] = srow

    @pl.when(i > 0)
    def _():
        g_ref[...] += g
        s_ref[...] += srow


def _run_moments(ps3, q16, tmm, k):
    bm = ps3.shape[0]
    w = 16 * k
    return pl.pallas_call(
        lambda a, b, c, d: _moments_kernel(a, b, c, d, k=k),
        grid=(bm // tmm,),
        in_specs=[
            pl.BlockSpec((tmm, k, 128), lambda i: (i, 0, 0)),
            pl.BlockSpec((tmm, 16), lambda i: (i, 0)),
        ],
        out_specs=[
            pl.BlockSpec((w, w), lambda i: (0, 0)),
            pl.BlockSpec((8, w), lambda i: (0, 0)),
        ],
        out_shape=[
            jax.ShapeDtypeStruct((w, w), jnp.float32),
            jax.ShapeDtypeStruct((8, w), jnp.float32),
        ],
    )(ps3, q16)


def _mlp_kernel(ps_ref, q_ref, w1q_ref, b1_ref, w2t_ref, wt1p_ref,
                bt1_ref, wt2t_ref, h2_ref, t2_ref, st_ref, *, k, mid):
    i = pl.program_id(0)
    tm2 = ps_ref.shape[0]
    ph3 = ps_ref[:, :, 0:16] - q_ref[...][:, None, :]

    phf = ph3.reshape(tm2 * k, 16)
    h1f = jax.lax.dot_general(phf, w1q_ref[...], (((1,), (0,)), ((), ())),
                              preferred_element_type=jnp.float32)
    h1f = _selu(h1f + b1_ref[...])
    h2f = jax.lax.dot_general(h1f, w2t_ref[...], (((1,), (0,)), ((), ())),
                              preferred_element_type=jnp.float32)
    h2_ref[...] = h2f.reshape(tm2, k, mid)

    t1 = bt1_ref[...]
    for j in range(k):
        t1 = t1 + jax.lax.dot_general(
            ph3[:, j, :], wt1p_ref[j], (((1,), (0,)), ((), ())),
            preferred_element_type=jnp.float32)
    t1 = _selu(t1)
    t2 = jax.lax.dot_general(t1, wt2t_ref[...], (((1,), (0,)), ((), ())),
                             preferred_element_type=jnp.float32)
    t2_ref[...] = t2

    kk = t2.shape[1]
    pad = jnp.zeros((1, kk), jnp.float32)
    row = jnp.concatenate([
        jnp.sum(h2f, axis=0, keepdims=True),
        jnp.sum(h2f * h2f, axis=0, keepdims=True),
        jnp.sum(t2, axis=0, keepdims=True),
        jnp.sum(t2 * t2, axis=0, keepdims=True),
        pad, pad, pad, pad], axis=0)

    @pl.when(i == 0)
    def _():
        st_ref[...] = row

    @pl.when(i > 0)
    def _():
        st_ref[...] += row


def _run_mlps(ps3, q16, w1q, b1e, w2t, wt1p, bt1e, wt2t, tm2, k, mid):
    bm = ps3.shape[0]
    kk = wt2t.shape[1]
    return pl.pallas_call(
        lambda *a: _mlp_kernel(*a, k=k, mid=mid),
        grid=(bm // tm2,),
        in_specs=[
            pl.BlockSpec((tm2, k, 128), lambda i: (i, 0, 0)),
            pl.BlockSpec((tm2, 16), lambda i: (i, 0)),
            pl.BlockSpec((16, mid), lambda i: (0, 0)),
            pl.BlockSpec((1, mid), lambda i: (0, 0)),
            pl.BlockSpec((mid, mid), lambda i: (0, 0)),
            pl.BlockSpec((k, 16, kk), lambda i: (0, 0, 0)),
            pl.BlockSpec((1, kk), lambda i: (0, 0)),
            pl.BlockSpec((kk, kk), lambda i: (0, 0)),
        ],
        out_specs=[
            pl.BlockSpec((tm2, k, mid), lambda i: (i, 0, 0)),
            pl.BlockSpec((tm2, kk), lambda i: (i, 0)),
            pl.BlockSpec((8, kk), lambda i: (0, 0)),
        ],
        out_shape=[
            jax.ShapeDtypeStruct((bm, k, mid), jnp.float32),
            jax.ShapeDtypeStruct((bm, kk), jnp.float32),
            jax.ShapeDtypeStruct((8, kk), jnp.float32),
        ],
    )(ps3, q16, w1q, b1e, w2t, wt1p, bt1e, wt2t)


def _final_kernel(h2_ref, t2_ref, xs_ref, p2_ref, wt3t_ref, wfr_ref,
                  out_ref, st_ref, *, k, cout):
    b = pl.program_id(0)
    i = pl.program_id(1)
    a2 = p2_ref[0:1, :]
    c2 = p2_ref[1:2, :]
    at2 = p2_ref[2:3, :]
    ct2 = p2_ref[3:4, :]

    hh = _selu(h2_ref[...] * a2[None] + c2[None])
    tt = _selu(t2_ref[...] * at2 + ct2)
    t3 = jax.lax.dot_general(tt, wt3t_ref[...], (((1,), (0,)), ((), ())),
                             preferred_element_type=jnp.float32)
    xh = jnp.concatenate([hh, xs_ref[...]], axis=2)
    tm3 = xh.shape[0]

    acc = jnp.zeros((cout, tm3), jnp.float32)
    for kk_ in range(k):
        xm = t3[:, k * kk_:k * kk_ + 1] * xh[:, 0, :]
        for j in range(1, k):
            xm = xm + t3[:, k * kk_ + j:k * kk_ + j + 1] * xh[:, j, :]
        acc = acc + jax.lax.dot_general(
            wfr_ref[kk_], xm, (((0,), (1,)), ((), ())),
            preferred_element_type=jnp.float32)
    out_ref[0] = acc

    row = jnp.concatenate([
        jnp.sum(acc, axis=1, keepdims=True),
        jnp.sum(acc * acc, axis=1, keepdims=True),
        jnp.zeros((cout, 6), jnp.float32)], axis=1)

    first = jnp.logical_and(b == 0, i == 0)

    @pl.when(first)
    def _():
        st_ref[...] = row

    @pl.when(jnp.logical_not(first))
    def _():
        st_ref[...] += row


def _run_final(h2raw, t2raw, x_sel3, p2, wt3t, wfr, bb, m, tm3, k, cout):
    cin = x_sel3.shape[2]
    mid = h2raw.shape[2]
    kk = t2raw.shape[1]
    nt = m // tm3
    return pl.pallas_call(
        lambda *a: _final_kernel(*a, k=k, cout=cout),
        grid=(bb, nt),
        in_specs=[
            pl.BlockSpec((tm3, k, mid), lambda b, i: (b * nt + i, 0, 0)),
            pl.BlockSpec((tm3, kk), lambda b, i: (b * nt + i, 0)),
            pl.BlockSpec((tm3, k, cin), lambda b, i: (b * nt + i, 0, 0)),
            pl.BlockSpec((8, kk), lambda b, i: (0, 0)),
            pl.BlockSpec((kk, kk), lambda b, i: (0, 0)),
            pl.BlockSpec((k, mid + cin, cout), lambda b, i: (0, 0, 0)),
        ],
        out_specs=[
            pl.BlockSpec((1, cout, tm3), lambda b, i: (b, 0, i)),
            pl.BlockSpec((cout, 8), lambda b, i: (0, 0)),
        ],
        out_shape=[
            jax.ShapeDtypeStruct((bb, cout, m), jnp.float32),
            jax.ShapeDtypeStruct((cout, 8), jnp.float32),
        ],
    )(h2raw, t2raw, x_sel3, p2, wt3t, wfr)


def _bnout_kernel(o_ref, pf_ref, out_ref):
    af = pf_ref[:, 0:1]
    cf = pf_ref[:, 1:2]
    out_ref[0] = _selu(o_ref[0] * af + cf)


def _run_bnout(oraw, pf, tm4):
    bb, cout, m = oraw.shape
    return pl.pallas_call(
        _bnout_kernel,
        grid=(bb, m // tm4),
        in_specs=[
            pl.BlockSpec((1, cout, tm4), lambda b, i: (b, 0, i)),
            pl.BlockSpec((cout, 8), lambda b, i: (0, 0)),
        ],
        out_specs=pl.BlockSpec((1, cout, tm4), lambda b, i: (b, 0, i)),
        out_shape=jax.ShapeDtypeStruct((bb, cout, m), jnp.float32),
    )(oraw, pf)


def kernel(p, x, q, W1, g1, b1, W2, g2, b2, Wt1, gt1, bt1, Wt2, gt2, bt2,
           Wt3, Wf, gf, bf):
    bb, _, n = p.shape
    m = q.shape[1]
    cin = x.shape[1]
    cout, _, _, k = Wf.shape
    mid = W1.shape[0]
    kk = Wt1.shape[0]
    bm = bb * m
    bmk = bm * k

    tm = min(256, m)
    tmm = min(1024, bm)
    tm2 = min(512, bm)
    tm3 = min(512, m)
    tm4 = min(1024, m)

    idx = _run_topk(p, q, tm, k)
    idx_flat = idx.reshape(1, bmk)

    ptp = jnp.pad(jnp.transpose(p, (0, 2, 1)),
                  ((0, 0), (0, 0), (0, 125))).reshape(bb * n, 128)
    xt = jnp.transpose(x, (0, 2, 1)).reshape(bb * n, cin)
    p_sel = _sc_gather(ptp, idx_flat, 128)
    x_sel = _sc_gather(xt, idx_flat, 128)
    ps3 = p_sel.reshape(bm, k, 128)
    x_sel3 = x_sel.reshape(bm, k, cin)
    q16 = jnp.pad(q.reshape(bm, 3), ((0, 0), (0, 13)))

    g128, s128 = _run_moments(ps3, q16, tmm, k)
    s128 = s128[0]
    g4 = g128.reshape(k, 16, k, 16)[:, 0:3, :, 0:3]
    s2d = s128.reshape(k, 16)[:, 0:3]

    w1m = W1.reshape(mid, 3)
    mu3 = jnp.sum(s2d, axis=0) / bmk
    s3 = jnp.einsum('iaib->ab', g4) / bmk
    mean1 = w1m @ mu3
    e2 = jnp.sum((w1m @ s3) * w1m, axis=1)
    var1 = jnp.maximum(e2 - mean1 * mean1, 0.0)
    a1 = g1 / jnp.sqrt(var1 + _EPS)
    c1 = b1 - a1 * mean1
    w1q = jnp.pad((w1m.T * a1[None, :]), ((0, 13), (0, 0)))
    b1e = c1.reshape(1, mid)

    wt1sq = Wt1[:, :, 0, :]
    wt1km = jnp.transpose(wt1sq, (0, 2, 1)).reshape(kk, 3 * k)
    mu24 = (s2d / bm).reshape(3 * k)
    m24 = jnp.transpose(g4, (0, 1, 2, 3)).reshape(k, 3, k, 3)
    m24 = jnp.reshape(m24, (3 * k, 3 * k)) / bm
    meant1 = wt1km @ mu24
    e2t = jnp.sum((wt1km @ m24) * wt1km, axis=1)
    vart1 = jnp.maximum(e2t - meant1 * meant1, 0.0)
    at1 = gt1 / jnp.sqrt(vart1 + _EPS)
    ct1 = bt1 - at1 * meant1
    wt1p = jnp.pad(jnp.transpose(wt1sq, (2, 1, 0)) * at1[None, None, :],
                   ((0, 0), (0, 13), (0, 0)))
    bt1e = ct1.reshape(1, kk)

    w2t = W2[:, :, 0, 0].T
    wt2t = Wt2[:, :, 0, 0].T
    h2raw, t2raw, st2 = _run_mlps(ps3, q16, w1q, b1e, w2t, wt1p,
                                  bt1e, wt2t, tm2, k, mid)

    mean2 = st2[0] / bmk
    var2 = jnp.maximum(st2[1] / bmk - mean2 * mean2, 0.0)
    a2 = g2 / jnp.sqrt(var2 + _EPS)
    c2 = b2 - a2 * mean2
    meant2 = st2[2] / bm
    vart2 = jnp.maximum(st2[3] / bm - meant2 * meant2, 0.0)
    at2 = gt2 / jnp.sqrt(vart2 + _EPS)
    ct2 = bt2 - at2 * meant2
    p2 = jnp.stack([
        jnp.pad(a2, (0, kk - mid)), jnp.pad(c2, (0, kk - mid)),
        at2, ct2,
        jnp.zeros((kk,)), jnp.zeros((kk,)), jnp.zeros((kk,)),
        jnp.zeros((kk,))], axis=0)
    p2 = p2.astype(jnp.float32)

    wt3t = Wt3[:, :, 0, 0].T
    wfr = jnp.transpose(Wf[:, :, 0, :], (2, 1, 0))
    oraw, stf = _run_final(h2raw, t2raw, x_sel3, p2, wt3t, wfr,
                           bb, m, tm3, k, cout)

    meanf = stf[:, 0] / bm
    varf = jnp.maximum(stf[:, 1] / bm - meanf * meanf, 0.0)
    af = gf / jnp.sqrt(varf + _EPS)
    cf = bf - af * meanf
    pf = jnp.concatenate([af.reshape(cout, 1), cf.reshape(cout, 1),
                          jnp.zeros((cout, 6), jnp.float32)], axis=1)

    out = _run_bnout(oraw, pf, tm4)
    q_out = jnp.transpose(q, (0, 2, 1))
    return (q_out, out)


# speedup vs baseline: 14.6039x; 1.0399x over previous
import jax
import jax.numpy as jnp
from jax.experimental import pallas as pl
from jax.experimental.pallas import tpu as pltpu
from jax.experimental.pallas import tpu_sc as plsc


_EPS = 1e-5
_SELU_ALPHA = 1.6732632423543772
_SELU_SCALE = 1.0507009873554805


def _selu(v):
    return _SELU_SCALE * jnp.where(v > 0, v, _SELU_ALPHA * (jnp.exp(v) - 1.0))


def _topk_kernel(p_ref, q_ref, idx_ref, *, n, k):
    b = pl.program_id(0)
    p = p_ref[0]
    q = q_ref[0]
    tm = q.shape[0]
    sp = jnp.sum(p * p, axis=0, keepdims=True)
    sq = jnp.sum(q * q, axis=1, keepdims=True)
    qb = q.astype(jnp.bfloat16)
    pb = p.astype(jnp.bfloat16)
    dot = jax.lax.dot_general(qb, pb, (((1,), (0,)), ((), ())),
                              preferred_element_type=jnp.float32)
    d = (sq + sp) - 2.0 * dot
    iota = jax.lax.broadcasted_iota(jnp.int32, d.shape, 1)
    iota_k = jax.lax.broadcasted_iota(jnp.int32, (tm, k), 1)
    base = b * n
    zk_i = jnp.zeros((tm, k), jnp.int32)

    def body(j, carry):
        dd, ia, am_prev = carry
        dd = jnp.where(iota == am_prev, jnp.float32(jnp.inf), dd)
        mv = jnp.min(dd, axis=1, keepdims=True)
        am = jnp.min(jnp.where(dd == mv, iota, jnp.int32(n)), axis=1,
                     keepdims=True)
        ia = jnp.where(iota_k == j, am + base, ia)
        return dd, ia, am

    _, ia, _ = jax.lax.fori_loop(
        0, k, body, (d, zk_i, jnp.full((tm, 1), -1, jnp.int32)))
    idx_ref[0] = ia


def _run_topk(p, q, tm, k):
    bb, _, n = p.shape
    m = q.shape[1]
    return pl.pallas_call(
        lambda pr, qr, ir: _topk_kernel(pr, qr, ir, n=n, k=k),
        grid=(bb, m // tm),
        in_specs=[
            pl.BlockSpec((1, 3, n), lambda b, i: (b, 0, 0)),
            pl.BlockSpec((1, tm, 3), lambda b, i: (b, i, 0)),
        ],
        out_specs=pl.BlockSpec((1, tm, k), lambda b, i: (b, i, 0)),
        out_shape=jax.ShapeDtypeStruct((bb, m, k), jnp.int32),
    )(p, q)


def _sc_gather(data, idx_flat, win):
    n_idx = idx_flat.shape[1]
    width = data.shape[1]
    mesh = plsc.VectorSubcoreMesh(core_axis_name="c", subcore_axis_name="s")

    @pl.kernel(out_type=jax.ShapeDtypeStruct((n_idx, width), data.dtype),
               mesh=mesh)
    def gk(x_hbm, i_hbm, o_hbm):
        def body(i_vmem, o_vmem):
            pltpu.sync_copy(x_hbm.at[i_vmem.at[0]], o_vmem)

        pltpu.emit_pipeline(
            body,
            grid=(n_idx // win,),
            in_specs=[pl.BlockSpec((1, win), index_map=lambda i: (0, i))],
            out_specs=[pl.BlockSpec((win, width), index_map=lambda i: (i, 0))],
            core_axis_name=("c", "s"),
            dimension_semantics=(pltpu.PARALLEL,),
        )(i_hbm, o_hbm)

    return gk(data, idx_flat)


def _moments_kernel(ps_ref, q_ref, g_ref, s_ref, *, k):
    i = pl.program_id(0)
    q16 = q_ref[...]
    cols = [ps_ref[:, j, 0:16] - q16 for j in range(k)]
    cat = jnp.concatenate(cols, axis=1)
    g = jax.lax.dot_general(cat, cat, (((0,), (0,)), ((), ())),
                            preferred_element_type=jnp.float32)
    s = jnp.sum(cat, axis=0, keepdims=True)
    srow = jnp.concatenate(
        [s, jnp.zeros((7, s.shape[1]), jnp.float32)], axis=0)

    @pl.when(i == 0)
    def _():
        g_ref[...] = g
        s_ref[...] = srow

    @pl.when(i > 0)
    def _():
        g_ref[...] += g
        s_ref[...] += srow


def _run_moments(ps3, q16, tmm, k):
    bm = ps3.shape[0]
    w = 16 * k
    return pl.pallas_call(
        lambda a, b, c, d: _moments_kernel(a, b, c, d, k=k),
        grid=(bm // tmm,),
        in_specs=[
            pl.BlockSpec((tmm, k, 128), lambda i: (i, 0, 0)),
            pl.BlockSpec((tmm, 16), lambda i: (i, 0)),
        ],
        out_specs=[
            pl.BlockSpec((w, w), lambda i: (0, 0)),
            pl.BlockSpec((8, w), lambda i: (0, 0)),
        ],
        out_shape=[
            jax.ShapeDtypeStruct((w, w), jnp.float32),
            jax.ShapeDtypeStruct((8, w), jnp.float32),
        ],
    )(ps3, q16)


def _mlp_kernel(ps_ref, q_ref, w1q_ref, b1_ref, w2t_ref, wt1p_ref,
                bt1_ref, wt2t_ref, h2_ref, t2_ref, st_ref, *, k, mid):
    i = pl.program_id(0)
    tm2 = ps_ref.shape[0]
    ph3 = ps_ref[:, :, 0:16] - q_ref[...][:, None, :]

    phf = ph3.reshape(tm2 * k, 16)
    h1f = jax.lax.dot_general(phf, w1q_ref[...], (((1,), (0,)), ((), ())),
                              preferred_element_type=jnp.float32)
    h1f = _selu(h1f + b1_ref[...])
    h2f = jax.lax.dot_general(h1f, w2t_ref[...], (((1,), (0,)), ((), ())),
                              preferred_element_type=jnp.float32)
    h2_ref[...] = h2f.reshape(tm2, k, mid)

    t1 = bt1_ref[...]
    for j in range(k):
        t1 = t1 + jax.lax.dot_general(
            ph3[:, j, :], wt1p_ref[j], (((1,), (0,)), ((), ())),
            preferred_element_type=jnp.float32)
    t1 = _selu(t1)
    t2 = jax.lax.dot_general(t1, wt2t_ref[...], (((1,), (0,)), ((), ())),
                             preferred_element_type=jnp.float32)
    t2_ref[...] = t2

    kk = t2.shape[1]
    pad = jnp.zeros((1, kk), jnp.float32)
    row = jnp.concatenate([
        jnp.sum(h2f, axis=0, keepdims=True),
        jnp.sum(h2f * h2f, axis=0, keepdims=True),
        jnp.sum(t2, axis=0, keepdims=True),
        jnp.sum(t2 * t2, axis=0, keepdims=True),
        pad, pad, pad, pad], axis=0)

    @pl.when(i == 0)
    def _():
        st_ref[...] = row

    @pl.when(i > 0)
    def _():
        st_ref[...] += row


def _run_mlps(ps3, q16, w1q, b1e, w2t, wt1p, bt1e, wt2t, tm2, k, mid):
    bm = ps3.shape[0]
    kk = wt2t.shape[1]
    return pl.pallas_call(
        lambda *a: _mlp_kernel(*a, k=k, mid=mid),
        grid=(bm // tm2,),
        in_specs=[
            pl.BlockSpec((tm2, k, 128), lambda i: (i, 0, 0)),
            pl.BlockSpec((tm2, 16), lambda i: (i, 0)),
            pl.BlockSpec((16, mid), lambda i: (0, 0)),
            pl.BlockSpec((1, mid), lambda i: (0, 0)),
            pl.BlockSpec((mid, mid), lambda i: (0, 0)),
            pl.BlockSpec((k, 16, kk), lambda i: (0, 0, 0)),
            pl.BlockSpec((1, kk), lambda i: (0, 0)),
            pl.BlockSpec((kk, kk), lambda i: (0, 0)),
        ],
        out_specs=[
            pl.BlockSpec((tm2, k, mid), lambda i: (i, 0, 0)),
            pl.BlockSpec((tm2, kk), lambda i: (i, 0)),
            pl.BlockSpec((8, kk), lambda i: (0, 0)),
        ],
        out_shape=[
            jax.ShapeDtypeStruct((bm, k, mid), jnp.float32),
            jax.ShapeDtypeStruct((bm, kk), jnp.float32),
            jax.ShapeDtypeStruct((8, kk), jnp.float32),
        ],
    )(ps3, q16, w1q, b1e, w2t, wt1p, bt1e, wt2t)


def _final_kernel(h2_ref, t2_ref, xs_ref, p2_ref, wt3t_ref, wfr_ref,
                  out_ref, st_ref, *, k, cout):
    b = pl.program_id(0)
    i = pl.program_id(1)
    a2 = p2_ref[0:1, :]
    c2 = p2_ref[1:2, :]
    at2 = p2_ref[2:3, :]
    ct2 = p2_ref[3:4, :]

    hh = _selu(h2_ref[...] * a2[None] + c2[None])
    tt = _selu(t2_ref[...] * at2 + ct2)
    t3 = jax.lax.dot_general(tt, wt3t_ref[...], (((1,), (0,)), ((), ())),
                             preferred_element_type=jnp.float32)
    xh = jnp.concatenate([hh, xs_ref[...]], axis=2)
    tm3 = xh.shape[0]

    acc = jnp.zeros((cout, tm3), jnp.float32)
    for kk_ in range(k):
        xm = t3[:, k * kk_:k * kk_ + 1] * xh[:, 0, :]
        for j in range(1, k):
            xm = xm + t3[:, k * kk_ + j:k * kk_ + j + 1] * xh[:, j, :]
        acc = acc + jax.lax.dot_general(
            wfr_ref[kk_], xm, (((0,), (1,)), ((), ())),
            preferred_element_type=jnp.float32)
    out_ref[0] = acc

    row = jnp.concatenate([
        jnp.sum(acc, axis=1, keepdims=True),
        jnp.sum(acc * acc, axis=1, keepdims=True),
        jnp.zeros((cout, 6), jnp.float32)], axis=1)

    first = jnp.logical_and(b == 0, i == 0)

    @pl.when(first)
    def _():
        st_ref[...] = row

    @pl.when(jnp.logical_not(first))
    def _():
        st_ref[...] += row


def _run_final(h2raw, t2raw, x_sel3, p2, wt3t, wfr, bb, m, tm3, k, cout):
    cin = x_sel3.shape[2]
    mid = h2raw.shape[2]
    kk = t2raw.shape[1]
    nt = m // tm3
    return pl.pallas_call(
        lambda *a: _final_kernel(*a, k=k, cout=cout),
        grid=(bb, nt),
        in_specs=[
            pl.BlockSpec((tm3, k, mid), lambda b, i: (b * nt + i, 0, 0)),
            pl.BlockSpec((tm3, kk), lambda b, i: (b * nt + i, 0)),
            pl.BlockSpec((tm3, k, cin), lambda b, i: (b * nt + i, 0, 0)),
            pl.BlockSpec((8, kk), lambda b, i: (0, 0)),
            pl.BlockSpec((kk, kk), lambda b, i: (0, 0)),
            pl.BlockSpec((k, mid + cin, cout), lambda b, i: (0, 0, 0)),
        ],
        out_specs=[
            pl.BlockSpec((1, cout, tm3), lambda b, i: (b, 0, i)),
            pl.BlockSpec((cout, 8), lambda b, i: (0, 0)),
        ],
        out_shape=[
            jax.ShapeDtypeStruct((bb, cout, m), jnp.float32),
            jax.ShapeDtypeStruct((cout, 8), jnp.float32),
        ],
    )(h2raw, t2raw, x_sel3, p2, wt3t, wfr)


def _bnout_kernel(o_ref, pf_ref, out_ref):
    af = pf_ref[:, 0:1]
    cf = pf_ref[:, 1:2]
    out_ref[0] = _selu(o_ref[0] * af + cf)


def _run_bnout(oraw, pf, tm4):
    bb, cout, m = oraw.shape
    return pl.pallas_call(
        _bnout_kernel,
        grid=(bb, m // tm4),
        in_specs=[
            pl.BlockSpec((1, cout, tm4), lambda b, i: (b, 0, i)),
            pl.BlockSpec((cout, 8), lambda b, i: (0, 0)),
        ],
        out_specs=pl.BlockSpec((1, cout, tm4), lambda b, i: (b, 0, i)),
        out_shape=jax.ShapeDtypeStruct((bb, cout, m), jnp.float32),
    )(oraw, pf)


def kernel(p, x, q, W1, g1, b1, W2, g2, b2, Wt1, gt1, bt1, Wt2, gt2, bt2,
           Wt3, Wf, gf, bf):
    bb, _, n = p.shape
    m = q.shape[1]
    cin = x.shape[1]
    cout, _, _, k = Wf.shape
    mid = W1.shape[0]
    kk = Wt1.shape[0]
    bm = bb * m
    bmk = bm * k

    tm = min(256, m)
    tmm = min(1024, bm)
    tm2 = min(512, bm)
    tm3 = min(512, m)
    tm4 = min(1024, m)

    idx = _run_topk(p, q, tm, k)
    idx_flat = idx.reshape(1, bmk)

    ptp = jnp.pad(jnp.transpose(p, (0, 2, 1)),
                  ((0, 0), (0, 0), (0, 125))).reshape(bb * n, 128)
    xt = jnp.transpose(x, (0, 2, 1)).reshape(bb * n, cin)
    p_sel = _sc_gather(ptp, idx_flat, 128)
    x_sel = _sc_gather(xt, idx_flat, 128)
    ps3 = p_sel.reshape(bm, k, 128)
    x_sel3 = x_sel.reshape(bm, k, cin)
    q16 = jnp.pad(q.reshape(bm, 3), ((0, 0), (0, 13)))

    g128, s128 = _run_moments(ps3, q16, tmm, k)
    s128 = s128[0]
    g4 = g128.reshape(k, 16, k, 16)[:, 0:3, :, 0:3]
    s2d = s128.reshape(k, 16)[:, 0:3]

    w1m = W1.reshape(mid, 3)
    mu3 = jnp.sum(s2d, axis=0) / bmk
    s3 = jnp.einsum('iaib->ab', g4) / bmk
    mean1 = w1m @ mu3
    e2 = jnp.sum((w1m @ s3) * w1m, axis=1)
    var1 = jnp.maximum(e2 - mean1 * mean1, 0.0)
    a1 = g1 / jnp.sqrt(var1 + _EPS)
    c1 = b1 - a1 * mean1
    w1q = jnp.pad((w1m.T * a1[None, :]), ((0, 13), (0, 0)))
    b1e = c1.reshape(1, mid)

    wt1sq = Wt1[:, :, 0, :]
    wt1km = jnp.transpose(wt1sq, (0, 2, 1)).reshape(kk, 3 * k)
    mu24 = (s2d / bm).reshape(3 * k)
    m24 = jnp.transpose(g4, (0, 1, 2, 3)).reshape(k, 3, k, 3)
    m24 = jnp.reshape(m24, (3 * k, 3 * k)) / bm
    meant1 = wt1km @ mu24
    e2t = jnp.sum((wt1km @ m24) * wt1km, axis=1)
    vart1 = jnp.maximum(e2t - meant1 * meant1, 0.0)
    at1 = gt1 / jnp.sqrt(vart1 + _EPS)
    ct1 = bt1 - at1 * meant1
    wt1p = jnp.pad(jnp.transpose(wt1sq, (2, 1, 0)) * at1[None, None, :],
                   ((0, 0), (0, 13), (0, 0)))
    bt1e = ct1.reshape(1, kk)

    w2t = W2[:, :, 0, 0].T
    wt2t = Wt2[:, :, 0, 0].T
    h2raw, t2raw, st2 = _run_mlps(ps3, q16, w1q, b1e, w2t, wt1p,
                                  bt1e, wt2t, tm2, k, mid)

    mean2 = st2[0] / bmk
    var2 = jnp.maximum(st2[1] / bmk - mean2 * mean2, 0.0)
    a2 = g2 / jnp.sqrt(var2 + _EPS)
    c2 = b2 - a2 * mean2
    meant2 = st2[2] / bm
    vart2 = jnp.maximum(st2[3] / bm - meant2 * meant2, 0.0)
    at2 = gt2 / jnp.sqrt(vart2 + _EPS)
    ct2 = bt2 - at2 * meant2
    p2 = jnp.stack([
        jnp.pad(a2, (0, kk - mid)), jnp.pad(c2, (0, kk - mid)),
        at2, ct2,
        jnp.zeros((kk,)), jnp.zeros((kk,)), jnp.zeros((kk,)),
        jnp.zeros((kk,))], axis=0)
    p2 = p2.astype(jnp.float32)

    wt3t = Wt3[:, :, 0, 0].T
    wfr = jnp.transpose(Wf[:, :, 0, :], (2, 1, 0))
    oraw, stf = _run_final(h2raw, t2raw, x_sel3, p2, wt3t, wfr,
                           bb, m, tm3, k, cout)

    meanf = stf[:, 0] / bm
    varf = jnp.maximum(stf[:, 1] / bm - meanf * meanf, 0.0)
    af = gf / jnp.sqrt(varf + _EPS)
    cf = bf - af * meanf
    pf = jnp.concatenate([af.reshape(cout, 1), cf.reshape(cout, 1),
                          jnp.zeros((cout, 6), jnp.float32)], axis=1)

    out = _run_bnout(oraw, pf, tm4)
    q_out = jnp.transpose(q, (0, 2, 1))
    return (q_out, out)
